# baseline (speedup 1.0000x reference)
import functools

import jax
import jax.numpy as jnp
from jax import lax
from jax.experimental import pallas as pl
from jax.experimental.pallas import tpu as pltpu
from jax.experimental.pallas import tpu_sc as plsc

D = 64
COND = 256
B = 4096
H = 20

NC = 2
NS = 16
L = 16
NW = NC * NS
BPW = B // NW
G = 5
NG = H // G


def _sc_pool(ids_t, table):
    mesh = plsc.VectorSubcoreMesh(core_axis_name="c", subcore_axis_name="s")

    @functools.partial(
        pl.kernel,
        mesh=mesh,
        out_type=jax.ShapeDtypeStruct((B, D), jnp.float32),
        compiler_params=pltpu.CompilerParams(use_tc_tiling_on_sc=False),
        scratch_types=[
            pltpu.VMEM((H, BPW), jnp.int32),
            pltpu.VMEM((G, BPW, D), jnp.float32),
            pltpu.VMEM((BPW, D), jnp.float32),
            pltpu.SemaphoreType.DMA,
        ],
    )
    def k(ids_hbm, table_hbm, out_hbm, ids_v, buf_v, acc_v, sem):
        cid = lax.axis_index("c")
        sid = lax.axis_index("s")
        wid = sid * NC + cid
        base = wid * BPW
        pltpu.sync_copy(ids_hbm.at[:, pl.ds(base, BPW)], ids_v)
        for g in range(NG):
            cps = [
                pltpu.async_copy(
                    table_hbm.at[ids_v.at[g * G + j]], buf_v.at[j], sem
                )
                for j in range(G)
            ]
            for cp in cps:
                cp.wait()

            def acc_body(i, _, g=g):
                for cc in range(D // L):
                    sl = pl.ds(cc * L, L)
                    v = buf_v[0, i, sl]
                    for j in range(1, G):
                        v = v + buf_v[j, i, sl]
                    if g == 0:
                        acc_v[i, sl] = v
                    else:
                        acc_v[i, sl] = acc_v[i, sl] + v
                return 0

            lax.fori_loop(0, BPW, acc_body, 0)
        pltpu.sync_copy(acc_v, out_hbm.at[pl.ds(base, BPW)])

    return k(ids_t, table)


def _tc_proj(pooled, W, b2):
    BM = 512

    def mm(p_ref, w_ref, b_ref, o_ref):
        o_ref[...] = (
            jnp.dot(
                p_ref[...] * (1.0 / H), w_ref[...],
                preferred_element_type=jnp.float32,
            )
            + b_ref[...]
        )

    return pl.pallas_call(
        mm,
        grid=(B // BM,),
        in_specs=[
            pl.BlockSpec((BM, D), lambda i: (i, 0)),
            pl.BlockSpec((D, COND), lambda i: (0, 0)),
            pl.BlockSpec((1, COND), lambda i: (0, 0)),
        ],
        out_specs=pl.BlockSpec((BM, COND), lambda i: (i, 0)),
        out_shape=jax.ShapeDtypeStruct((B, COND), jnp.float32),
    )(pooled, W, b2)


def kernel(token_ids, table, W, b):
    ids_t = token_ids.T.astype(jnp.int32)
    pooled = _sc_pool(ids_t, table)
    return _tc_proj(pooled, W, b.reshape(1, COND))


# speedup vs baseline: 1.0003x; 1.0003x over previous
import functools

import jax
import jax.numpy as jnp
from jax import lax
from jax.experimental import pallas as pl
from jax.experimental.pallas import tpu as pltpu
from jax.experimental.pallas import tpu_sc as plsc

D = 64
COND = 256
B = 4096
H = 20

NC = 2
NS = 16
L = 16
NW = NC * NS
BPW = B // NW
IPW = BPW * H
SL = 128
NSL = IPW // SL
G = 5
NG = NSL // G
BPG = G * SL // H


def _sc_pool(ids_flat, table):
    mesh = plsc.VectorSubcoreMesh(core_axis_name="c", subcore_axis_name="s")

    @functools.partial(
        pl.kernel,
        mesh=mesh,
        out_type=jax.ShapeDtypeStruct((B, D), jnp.float32),
        compiler_params=pltpu.CompilerParams(use_tc_tiling_on_sc=False),
        scratch_types=[
            pltpu.VMEM((NSL, SL), jnp.int32),
            pltpu.VMEM((G * SL, D), jnp.float32),
            pltpu.VMEM((BPW, D), jnp.float32),
            pltpu.SemaphoreType.DMA,
        ],
    )
    def k(ids_hbm, table_hbm, out_hbm, ids_v, buf_v, pool_v, sem):
        cid = lax.axis_index("c")
        sid = lax.axis_index("s")
        wid = sid * NC + cid
        base = wid * BPW
        pltpu.sync_copy(ids_hbm.at[pl.ds(wid * NSL, NSL), :], ids_v)
        for g in range(NG):
            cps = [
                pltpu.async_copy(
                    table_hbm.at[ids_v.at[g * G + j]],
                    buf_v.at[pl.ds(j * SL, SL), :],
                    sem,
                )
                for j in range(G)
            ]
            for cp in cps:
                cp.wait()

            def pool_body(bi, _, g=g):
                r0 = bi * H
                for cc in range(D // L):
                    sl = pl.ds(cc * L, L)
                    t = [buf_v[r0 + j, sl] for j in range(H)]
                    while len(t) > 1:
                        t = [
                            t[i] + t[i + 1] if i + 1 < len(t) else t[i]
                            for i in range(0, len(t), 2)
                        ]
                    pool_v[g * BPG + bi, sl] = t[0]
                return 0

            lax.fori_loop(0, BPG, pool_body, 0)
        pltpu.sync_copy(pool_v, out_hbm.at[pl.ds(base, BPW)])

    return k(ids_flat, table)


def _tc_proj(pooled, W, b2):
    BM = 512

    def mm(p_ref, w_ref, b_ref, o_ref):
        o_ref[...] = (
            jnp.dot(
                p_ref[...] * (1.0 / H), w_ref[...],
                preferred_element_type=jnp.float32,
            )
            + b_ref[...]
        )

    return pl.pallas_call(
        mm,
        grid=(B // BM,),
        in_specs=[
            pl.BlockSpec((BM, D), lambda i: (i, 0)),
            pl.BlockSpec((D, COND), lambda i: (0, 0)),
            pl.BlockSpec((1, COND), lambda i: (0, 0)),
        ],
        out_specs=pl.BlockSpec((BM, COND), lambda i: (i, 0)),
        out_shape=jax.ShapeDtypeStruct((B, COND), jnp.float32),
    )(pooled, W, b2)


def kernel(token_ids, table, W, b):
    ids_flat = token_ids.reshape(B * H // SL, SL).astype(jnp.int32)
    pooled = _sc_pool(ids_flat, table)
    return _tc_proj(pooled, W, b.reshape(1, COND))
